# baseline (speedup 1.0000x reference)
import functools

import jax
import jax.numpy as jnp
from jax import lax
from jax.experimental import pallas as pl
from jax.experimental.pallas import tpu as pltpu
from jax.experimental.pallas import tpu_sc as plsc

_VOCAB = 100000
_EMBED = 64
_PAD = 128
_SCALE = 8.0

_B = 4096
_T = 200

_NW = 32
_BPW = _B // _NW
_NG = 5
_NT = 2

_PREP_ROWS = 2000


def _prep_body(w_ref, o_ref):
    base = pl.program_id(0) * _PREP_ROWS
    rid = lax.broadcasted_iota(jnp.int32, (_PREP_ROWS, _EMBED), 0) + base
    o_ref[:, : _EMBED] = jnp.where(rid == 0, 0.0, w_ref[...] * _SCALE)


def _prep_table(w):
    return pl.pallas_call(
        _prep_body,
        grid=(_VOCAB // _PREP_ROWS,),
        in_specs=[pl.BlockSpec((_PREP_ROWS, _EMBED), lambda i: (i, 0))],
        out_specs=pl.BlockSpec((_PREP_ROWS, _PAD), lambda i: (i, 0)),
        out_shape=jax.ShapeDtypeStruct((_VOCAB, _PAD), jnp.float32),
    )(w)


_mesh = plsc.VectorSubcoreMesh(core_axis_name="c", subcore_axis_name="s")


@functools.partial(
    pl.kernel,
    mesh=_mesh,
    out_type=jax.ShapeDtypeStruct((_T, _EMBED, _B), jnp.float32),
    scratch_types=[
        pltpu.VMEM((_T, _BPW), jnp.int32),
        pltpu.VMEM((_NG, _BPW, _PAD), jnp.float32),
        pltpu.VMEM((_NT, _EMBED, _BPW), jnp.float32),
    ]
    + [pltpu.SemaphoreType.DMA] * (_NG + _NT),
    compiler_params=pltpu.CompilerParams(
        use_tc_tiling_on_sc=True, needs_layout_passes=False
    ),
)
def _sc_gather(idx_hbm, table_hbm, out_hbm, idx_v, gbuf, tbuf, *sems):
    gsem, osem = sems[:_NG], sems[_NG:]
    rows16 = [lax.iota(jnp.int32, 16) + 16 * j for j in range(8)]
    wid = lax.axis_index("s") * 2 + lax.axis_index("c")
    bcol = wid * _BPW
    pltpu.sync_copy(idx_hbm.at[:, pl.ds(bcol, _BPW)], idx_v)

    def _issue_gather(t, g):
        pltpu.async_copy(table_hbm.at[idx_v.at[t]], gbuf.at[g], gsem[g])

    def _wait_gather(g):
        pltpu.make_async_copy(
            table_hbm.at[idx_v.at[0]], gbuf.at[g], gsem[g]
        ).wait()

    def _issue_out(t, tb):
        pltpu.async_copy(
            tbuf.at[tb], out_hbm.at[t, :, pl.ds(bcol, _BPW)], osem[tb]
        )

    def _wait_out(tb):
        pltpu.make_async_copy(
            tbuf.at[tb], out_hbm.at[0, :, pl.ds(bcol, _BPW)], osem[tb]
        ).wait()

    def _transpose(g, tb):
        @plsc.parallel_loop(0, _EMBED, unroll=4)
        def erow(e0):
            for j in range(8):
                ev = jnp.bitwise_and(rows16[j] + e0, _EMBED - 1)
                v = plsc.load_gather(gbuf.at[g], [rows16[j], ev])
                plsc.store_scatter(tbuf.at[tb], [ev, rows16[j]], v)

    def _step(t, g, tb, wait_out, issue_next):
        if wait_out:
            _wait_out(tb)
        _wait_gather(g)
        _transpose(g, tb)
        _issue_out(t, tb)
        if issue_next:
            _issue_gather(t + _NG, g)

    for g in range(_NG):
        _issue_gather(g, g)

    for u in range(_NG):
        _step(u, u, u % _NT, u >= _NT, True)

    def outer(s, carry):
        t0 = s * _NG
        for u in range(_NG):
            _step(t0 + u, u, u % _NT, True, True)
        return carry

    lax.fori_loop(1, _T // _NG - 1, outer, 0)

    t_last = _T - _NG
    for u in range(_NG):
        _step(t_last + u, u, u % _NT, True, False)

    for tb in range(_NT):
        _wait_out(tb)


def kernel(inputs, shared_weights):
    table = _prep_table(shared_weights)
    idx_t = jnp.transpose(inputs.astype(jnp.int32))
    out = _sc_gather(idx_t, table)
    return jnp.transpose(out, (2, 0, 1))


# speedup vs baseline: 1.0767x; 1.0767x over previous
import functools

import jax
import jax.numpy as jnp
from jax import lax
from jax.experimental import pallas as pl
from jax.experimental.pallas import tpu as pltpu
from jax.experimental.pallas import tpu_sc as plsc

_VOCAB = 100000
_EMBED = 64
_PAIR = 128
_SCALE = 8.0

_B = 4096
_T = 200

_NW = 32
_BPW = _B // _NW
_NG = 4
_NT = 2
_NH = _NG + 2

_mesh = plsc.VectorSubcoreMesh(core_axis_name="c", subcore_axis_name="s")


@functools.partial(
    pl.kernel,
    mesh=_mesh,
    out_type=jax.ShapeDtypeStruct((_T, _EMBED, _B), jnp.float32),
    scratch_types=[
        pltpu.VMEM((_T, _BPW), jnp.int32),
        pltpu.VMEM((_NH, _BPW), jnp.int32),
        pltpu.VMEM((_NG, _BPW, _PAIR), jnp.float32),
        pltpu.VMEM((_NT, _EMBED, _BPW), jnp.float32),
    ]
    + [pltpu.SemaphoreType.DMA] * (_NG + _NT),
    compiler_params=pltpu.CompilerParams(
        use_tc_tiling_on_sc=True, needs_layout_passes=False
    ),
)
def _sc_gather(idx_hbm, table_hbm, out_hbm, idx_v, idxh_v, gbuf, tbuf, *sems):
    gsem, osem = sems[:_NG], sems[_NG:]
    rows16 = [lax.iota(jnp.int32, 16) + 16 * j for j in range(8)]
    wid = lax.axis_index("s") * 2 + lax.axis_index("c")
    bcol = wid * _BPW
    pltpu.sync_copy(idx_hbm.at[:, pl.ds(bcol, _BPW)], idx_v)

    def _issue_gather(t, g):
        h = t % _NH
        for j in range(8):
            sl = pl.ds(16 * j, 16)
            idxh_v[h, sl] = jnp.right_shift(idx_v[t, sl], 1)
        pltpu.async_copy(table_hbm.at[idxh_v.at[h]], gbuf.at[g], gsem[g])

    def _wait_gather(g):
        pltpu.make_async_copy(
            table_hbm.at[idxh_v.at[0]], gbuf.at[g], gsem[g]
        ).wait()

    def _issue_out(t, tb):
        pltpu.async_copy(
            tbuf.at[tb], out_hbm.at[t, :, pl.ds(bcol, _BPW)], osem[tb]
        )

    def _wait_out(tb):
        pltpu.make_async_copy(
            tbuf.at[tb], out_hbm.at[0, :, pl.ds(bcol, _BPW)], osem[tb]
        ).wait()

    def _transpose(t, g, tb):
        half = []
        scale = []
        for j in range(8):
            iv = idx_v[t, pl.ds(16 * j, 16)]
            half.append(jnp.left_shift(jnp.bitwise_and(iv, 1), 6))
            scale.append(jnp.where(iv == 0, 0.0, _SCALE))

        @plsc.parallel_loop(0, _EMBED, unroll=4)
        def erow(e0):
            for j in range(8):
                ev = jnp.bitwise_and(rows16[j] + e0, _EMBED - 1)
                v = plsc.load_gather(gbuf.at[g], [rows16[j], half[j] + ev])
                plsc.store_scatter(tbuf.at[tb], [ev, rows16[j]], v * scale[j])

    def _step(t, g, tb, wait_out, issue_next):
        if wait_out:
            _wait_out(tb)
        _wait_gather(g)
        _transpose(t, g, tb)
        _issue_out(t, tb)
        if issue_next:
            _issue_gather(t + _NG, g)

    for g in range(_NG):
        _issue_gather(g, g)

    for u in range(_NG):
        _step(u, u, u % _NT, u >= _NT, True)

    def outer(s, carry):
        t0 = s * _NG
        for u in range(_NG):
            _step(t0 + u, u, u % _NT, True, True)
        return carry

    lax.fori_loop(1, _T // _NG - 1, outer, 0)

    t_last = _T - _NG
    for u in range(_NG):
        _step(t_last + u, u, u % _NT, True, False)

    for tb in range(_NT):
        _wait_out(tb)


def kernel(inputs, shared_weights):
    pairs = shared_weights.reshape(_VOCAB // 2, _PAIR)
    idx_t = jnp.transpose(inputs.astype(jnp.int32))
    out = _sc_gather(idx_t, pairs)
    return jnp.transpose(out, (2, 0, 1))
